# baseline (speedup 1.0000x reference)
import functools

import jax
import jax.numpy as jnp
from jax import lax
from jax.experimental import pallas as pl
from jax.experimental.pallas import tpu as pltpu
from jax.experimental.pallas import tpu_sc as plsc

_VOCAB = 1_000_000
_DIM = 64
_BATCH = 16384
_PAIRED_ROWS = _VOCAB // 2
_PAIRED_DIM = 2 * _DIM

_NUM_CORES = 2
_NUM_SUBCORES = 16
_NUM_WORKERS = _NUM_CORES * _NUM_SUBCORES
_B_PER_W = _BATCH // _NUM_WORKERS
_CHUNK = 128
_N_CHUNKS = _B_PER_W // _CHUNK


def _gather_body(idx_hbm, table2_hbm, out2_hbm, idx_v, rows_v, sem):
    wid = lax.axis_index("s") * _NUM_CORES + lax.axis_index("c")
    base = wid * _B_PER_W

    for ch in range(_N_CHUNKS):
        pltpu.sync_copy(idx_hbm.at[pl.ds(base + ch * _CHUNK, _CHUNK)],
                        idx_v.at[ch])

    copies = [
        pltpu.async_copy(table2_hbm.at[idx_v.at[ch]],
                         rows_v.at[pl.ds(ch * _CHUNK, _CHUNK), :], sem)
        for ch in range(_N_CHUNKS)
    ]
    for c in copies:
        c.wait()

    pltpu.sync_copy(rows_v, out2_hbm.at[pl.ds(base, _B_PER_W)])


_gather_pairs = functools.partial(
    pl.kernel,
    mesh=plsc.VectorSubcoreMesh(core_axis_name="c", subcore_axis_name="s"),
    out_type=jax.ShapeDtypeStruct((_BATCH, _PAIRED_DIM), jnp.float32),
    scratch_types=[
        pltpu.VMEM((_N_CHUNKS, _CHUNK), jnp.int32),
        pltpu.VMEM((_B_PER_W, _PAIRED_DIM), jnp.float32),
        pltpu.SemaphoreType.DMA,
    ],
    compiler_params=pltpu.CompilerParams(skip_device_barrier=True),
)(_gather_body)


@jax.jit
def kernel(labels, table):
    labels = labels.astype(jnp.int32)
    table2 = table.reshape(_PAIRED_ROWS, _PAIRED_DIM)
    pairs = _gather_pairs(labels >> 1, table2)
    odd = (labels & 1)[:, None] == 1
    return jnp.where(odd, pairs[:, _DIM:], pairs[:, :_DIM])


# speedup vs baseline: 1.0080x; 1.0080x over previous
import functools

import jax
import jax.numpy as jnp
from jax import lax
from jax.experimental import pallas as pl
from jax.experimental.pallas import tpu as pltpu
from jax.experimental.pallas import tpu_sc as plsc

_VOCAB = 1_000_000
_DIM = 64
_BATCH = 16384

_NUM_CORES = 2
_NUM_SUBCORES = 16
_NUM_WORKERS = _NUM_CORES * _NUM_SUBCORES
_B_PER_W = _BATCH // _NUM_WORKERS
_CHUNK = 128
_N_CHUNKS = _B_PER_W // _CHUNK


def _embed_body(labels_hbm, table_hbm, out_hbm, idx_v, rows_v, sem):
    wid = lax.axis_index("s") * _NUM_CORES + lax.axis_index("c")
    base = wid * _B_PER_W

    for j in range(_N_CHUNKS):
        pltpu.sync_copy(labels_hbm.at[pl.ds(base + j * _CHUNK, _CHUNK)],
                        idx_v.at[j])

    copies = [
        pltpu.async_copy(table_hbm.at[idx_v.at[j]],
                         rows_v.at[pl.ds(j * _CHUNK, _CHUNK)], sem)
        for j in range(_N_CHUNKS)
    ]
    for c in copies:
        c.wait()

    pltpu.sync_copy(rows_v, out_hbm.at[pl.ds(base, _B_PER_W)])


_embed_lookup = functools.partial(
    pl.kernel,
    mesh=plsc.VectorSubcoreMesh(core_axis_name="c", subcore_axis_name="s"),
    out_type=jax.ShapeDtypeStruct((_BATCH, _DIM), jnp.float32),
    scratch_types=[
        pltpu.VMEM((_N_CHUNKS, _CHUNK), jnp.int32),
        pltpu.VMEM((_B_PER_W, _DIM), jnp.float32),
        pltpu.SemaphoreType.DMA,
    ],
    compiler_params=pltpu.CompilerParams(
        use_tc_tiling_on_sc=False, skip_device_barrier=True),
)(_embed_body)


@jax.jit
def kernel(labels, table):
    return _embed_lookup(labels.astype(jnp.int32), table)


# speedup vs baseline: 1.6631x; 1.6498x over previous
import functools

import jax
import jax.numpy as jnp
from jax import lax
from jax.experimental import pallas as pl
from jax.experimental.pallas import tpu as pltpu
from jax.experimental.pallas import tpu_sc as plsc

_VOCAB = 1_000_000
_DIM = 64
_BATCH = 16384

_NUM_CORES = 2
_NUM_SUBCORES = 16
_NUM_WORKERS = _NUM_CORES * _NUM_SUBCORES
_B_PER_W = _BATCH // _NUM_WORKERS
_GROUP = 16
_N_GROUPS = _B_PER_W // _GROUP


def _embed_body(labels_hbm, table_hbm, out_hbm, idx_v, rows_v, sem):
    wid = lax.axis_index("s") * _NUM_CORES + lax.axis_index("c")
    base = wid * _B_PER_W

    pltpu.sync_copy(labels_hbm.at[pl.ds(base, _B_PER_W)], idx_v)

    def wave(g, carry):
        gbase = g * _GROUP
        vec = idx_v[pl.ds(gbase, _GROUP)]
        copies = []
        for j in range(_GROUP):
            row = vec[j]
            copies.append(
                pltpu.async_copy(table_hbm.at[pl.ds(row, 1), :],
                                 rows_v.at[pl.ds(gbase + j, 1), :], sem))
        for c in copies:
            c.wait()
        return carry

    lax.fori_loop(0, _N_GROUPS, wave, 0)

    pltpu.sync_copy(rows_v, out_hbm.at[pl.ds(base, _B_PER_W)])


_embed_lookup = functools.partial(
    pl.kernel,
    mesh=plsc.VectorSubcoreMesh(core_axis_name="c", subcore_axis_name="s"),
    out_type=jax.ShapeDtypeStruct((_BATCH, _DIM), jnp.float32),
    scratch_types=[
        pltpu.VMEM((_B_PER_W,), jnp.int32),
        pltpu.VMEM((_B_PER_W, _DIM), jnp.float32),
        pltpu.SemaphoreType.DMA,
    ],
    compiler_params=pltpu.CompilerParams(skip_device_barrier=True),
)(_embed_body)


@jax.jit
def kernel(labels, table):
    return _embed_lookup(labels.astype(jnp.int32), table)
